# Initial kernel scaffold; baseline (speedup 1.0000x reference)
import jax
import jax.numpy as jnp
from jax.experimental import pallas as pl


def kernel(inputs, position_table):
    raise NotImplementedError("write your pallas kernel here")


# speedup vs baseline: 1.7295x; 1.7295x over previous
import jax
import jax.numpy as jnp
from jax.experimental import pallas as pl

SEQ_BLOCK = 256


def _add_kernel(x_ref, t_ref, o_ref):
    o_ref[...] = x_ref[...] + t_ref[...][None, :, :]


def kernel(inputs, position_table):
    batch, seq, d = inputs.shape
    grid = (seq // SEQ_BLOCK,)
    return pl.pallas_call(
        _add_kernel,
        grid=grid,
        in_specs=[
            pl.BlockSpec((batch, SEQ_BLOCK, d), lambda i: (0, i, 0)),
            pl.BlockSpec((SEQ_BLOCK, d), lambda i: (i, 0)),
        ],
        out_specs=pl.BlockSpec((batch, SEQ_BLOCK, d), lambda i: (0, i, 0)),
        out_shape=jax.ShapeDtypeStruct((batch, seq, d), inputs.dtype),
    )(inputs, position_table)


# speedup vs baseline: 1.7360x; 1.0037x over previous
import jax
import jax.numpy as jnp
from jax.experimental import pallas as pl

SEQ_BLOCK = 1024


def _add_kernel(x_ref, t_ref, o_ref):
    o_ref[...] = x_ref[...] + t_ref[...][None, :, :]


def kernel(inputs, position_table):
    batch, seq, d = inputs.shape
    grid = (seq // SEQ_BLOCK, batch)
    return pl.pallas_call(
        _add_kernel,
        grid=grid,
        in_specs=[
            pl.BlockSpec((1, SEQ_BLOCK, d), lambda i, j: (j, i, 0)),
            pl.BlockSpec((SEQ_BLOCK, d), lambda i, j: (i, 0)),
        ],
        out_specs=pl.BlockSpec((1, SEQ_BLOCK, d), lambda i, j: (j, i, 0)),
        out_shape=jax.ShapeDtypeStruct((batch, seq, d), inputs.dtype),
    )(inputs, position_table)
